# baseline (speedup 1.0000x reference)
import jax
import jax.numpy as jnp
from jax.experimental import pallas as pl

NUM_BLOCKS = 4
TOKEN_DIM = 512
TOKEN_CLASS = 2048
TOKEN_NUM = 34
CB_TILE = 512
RBLK = 2176

_BF16 = jnp.bfloat16
_F32 = jnp.float32


def _ln(x, g, b, eps=1e-5):
    m = jnp.mean(x, axis=-1, keepdims=True)
    v = jnp.mean((x - m) ** 2, axis=-1, keepdims=True)
    return (x - m) / jnp.sqrt(v + eps) * g + b


def _mlp(x, W1, b1, W2, b2):
    h = jax.nn.gelu(x @ W1 + b1, approximate=False)
    return h @ W2 + b2


def _mixer(x, p, i):
    y = _ln(x, p['ln1_g'][i], p['ln1_b'][i])
    y = jnp.swapaxes(y, 1, 2)
    y = _mlp(y, p['tW1'][i], p['tb1'][i], p['tW2'][i], p['tb2'][i])
    y = jnp.swapaxes(y, 1, 2)
    z = _ln(x + y, p['ln2_g'][i], p['ln2_b'][i])
    z = _mlp(z, p['cW1'][i], p['cb1'][i], p['cW2'][i], p['cb2'][i])
    return x + y + z


def _vq_kernel(enc_ref, cbsq_ref, cbt16_ref, cb16_ref,
               idx_ref, quant_ref, elat_ref):
    step = pl.program_id(0)
    rows = enc_ref.shape[0]
    enc = enc_ref[...]
    enc16 = enc.astype(_BF16)
    enc_sq = jnp.sum(enc * enc, axis=1, keepdims=True)
    best = jnp.full((rows, 1), jnp.inf, _F32)
    bidx = jnp.zeros((rows, 1), jnp.int32)
    n_tiles = TOKEN_CLASS // CB_TILE
    for t in range(n_tiles):
        cb_sq = cbsq_ref[:, t * CB_TILE:(t + 1) * CB_TILE]
        sc = jax.lax.dot_general(
            enc16, cbt16_ref[:, t * CB_TILE:(t + 1) * CB_TILE],
            (((1,), (0,)), ((), ())), preferred_element_type=_F32)
        d = enc_sq + cb_sq - 2.0 * sc
        m = jnp.min(d, axis=1, keepdims=True)
        col = jax.lax.broadcasted_iota(jnp.int32, (rows, CB_TILE), 1)
        a = jnp.min(jnp.where(d == m, col + t * CB_TILE, TOKEN_CLASS),
                    axis=1, keepdims=True)
        upd = m < best
        best = jnp.where(upd, m, best)
        bidx = jnp.where(upd, a, bidx)
    idx_ref[...] = bidx
    quant = jnp.zeros((rows, TOKEN_DIM), _F32)
    for t in range(n_tiles):
        col = jax.lax.broadcasted_iota(jnp.int32, (rows, CB_TILE), 1)
        oh = (bidx == col + t * CB_TILE).astype(_BF16)
        quant = quant + jax.lax.dot_general(
            oh, cb16_ref[t * CB_TILE:(t + 1) * CB_TILE, :],
            (((1,), (0,)), ((), ())), preferred_element_type=_F32)
    quant_ref[...] = quant
    psum = jnp.sum((quant - enc) ** 2)

    @pl.when(step == 0)
    def _init():
        elat_ref[...] = jnp.zeros_like(elat_ref)

    elat_ref[0, 0, :] = elat_ref[0, 0, :] + psum


def kernel(pose, params, codebook):
    bs = pose.shape[0]
    f = pose @ params['W_start'] + params['b_start']
    for i in range(NUM_BLOCKS):
        f = _mixer(f, params['enc'], i)
    f = _ln(f, params['enc_lnf_g'], params['enc_lnf_b'])
    f = jnp.swapaxes(f, 1, 2)
    f = f @ params['W_tok'] + params['b_tok']
    f = jnp.swapaxes(f, 1, 2)
    f = f @ params['W_feat'] + params['b_feat']
    enc = f.reshape(-1, TOKEN_DIM)
    n_rows = enc.shape[0]
    n_steps = n_rows // RBLK

    cs = lambda x: pl.BlockSpec(x.shape, lambda i, _n=x.ndim: (0,) * _n)
    idx2d, quant, elat = pl.pallas_call(
        _vq_kernel,
        grid=(n_steps,),
        in_specs=[pl.BlockSpec((RBLK, TOKEN_DIM), lambda i: (i, 0)),
                  cs(jnp.zeros((1, TOKEN_CLASS))), cs(codebook.T),
                  cs(codebook)],
        out_specs=(pl.BlockSpec((RBLK, 1), lambda i: (i, 0)),
                   pl.BlockSpec((RBLK, TOKEN_DIM), lambda i: (i, 0)),
                   pl.BlockSpec((1, 1, 128), lambda i: (0, 0, 0))),
        out_shape=(jax.ShapeDtypeStruct((n_rows, 1), jnp.int32),
                   jax.ShapeDtypeStruct((n_rows, TOKEN_DIM), _F32),
                   jax.ShapeDtypeStruct((1, 1, 128), _F32)),
    )(enc, jnp.sum(codebook ** 2, axis=1).reshape(1, TOKEN_CLASS),
      codebook.T.astype(_BF16), codebook.astype(_BF16))

    idx = idx2d.reshape(n_rows)
    e_latent_loss = elat[0, 0, 0] / jnp.float32(n_rows * TOKEN_DIM)
    p = quant.reshape(bs, -1, TOKEN_DIM)
    p = jnp.swapaxes(p, 1, 2)
    p = p @ params['W_dtok'] + params['b_dtok']
    p = jnp.swapaxes(p, 1, 2)
    df = p @ params['W_dstart'] + params['b_dstart']
    for i in range(NUM_BLOCKS):
        df = _mixer(df, params['dec'], i)
    df = _ln(df, params['dec_lnf_g'], params['dec_lnf_b'])
    rec = df @ params['W_rec'] + params['b_rec']
    return rec, idx, e_latent_loss


# speedup vs baseline: 1.4234x; 1.4234x over previous
import jax
import jax.numpy as jnp
from jax.experimental import pallas as pl

NUM_JOINTS = 24
INPUT_DIM = 9
HID = 512
N_MIX = 4
TOKEN_NUM = 34
TOKEN_CLASS = 2048
TOKEN_DIM = 512
CB_TILE = 512

BBLK = 32

_BF16 = jnp.bfloat16
_F32 = jnp.float32


def _mm(a, b16):
    return jax.lax.dot_general(
        a.astype(_BF16) if a.dtype != _BF16 else a, b16,
        (((1,), (0,)), ((), ())), preferred_element_type=_F32)


def _ln(x, g, b, eps=1e-5):
    m = jnp.mean(x, axis=-1, keepdims=True)
    v = jnp.mean((x - m) ** 2, axis=-1, keepdims=True)
    return (x - m) / jnp.sqrt(v + eps) * g + b


def _gelu(x):
    return 0.5 * x * (1.0 + jax.lax.erf(x * jnp.float32(0.7071067811865476)))


def _token_mix(y2d, bblk, t_in, w1, b1, w2, b2):
    y3 = y2d.reshape(bblk, t_in, HID)
    yt = jnp.swapaxes(y3, 1, 2).reshape(bblk * HID, t_in)
    h = _gelu(_mm(yt, w1) + b1)
    z = _mm(h, w2) + b2
    t_out = z.shape[-1]
    z3 = z.reshape(bblk, HID, t_out)
    return jnp.swapaxes(z3, 1, 2).reshape(bblk * t_out, HID)


def _mixer(x2d, bblk, t, ln1_g, ln1_b, tw1, tb1, tw2, tb2,
           ln2_g, ln2_b, cw1, cb1, cw2, cb2):
    y = _ln(x2d, ln1_g, ln1_b)
    y = _token_mix(y, bblk, t, tw1, tb1, tw2, tb2)
    x1 = x2d + y
    z = _ln(x1, ln2_g, ln2_b)
    z = _mm(z, cw1) + cb1
    z = _gelu(z)
    z = _mm(z, cw2) + cb2
    return x1 + z


def _vqdec_kernel(
    enc_ref, cbsq_ref, cbt16_ref, cb16_ref,
    w_dtok_ref, b_dtok_ref, w_dstart_ref, b_dstart_ref,
    dec_ln1g_ref, dec_ln1b_ref, dec_tw1_ref, dec_tb1_ref, dec_tw2_ref,
    dec_tb2_ref, dec_ln2g_ref, dec_ln2b_ref, dec_cw1_ref, dec_cb1_ref,
    dec_cw2_ref, dec_cb2_ref,
    dec_lnfg_ref, dec_lnfb_ref,
    w_rec_ref, b_rec_ref,
    rec_ref, idx_ref, elat_ref,
):
    step = pl.program_id(0)
    rows = enc_ref.shape[0]
    bblk = rows // TOKEN_NUM

    enc = enc_ref[...]
    enc16 = enc.astype(_BF16)
    enc_sq = jnp.sum(enc * enc, axis=1, keepdims=True)
    best = jnp.full((rows, 1), jnp.inf, _F32)
    bidx = jnp.zeros((rows, 1), jnp.int32)
    n_tiles = TOKEN_CLASS // CB_TILE
    for t in range(n_tiles):
        cb_sq = cbsq_ref[:, t * CB_TILE:(t + 1) * CB_TILE]
        sc = jax.lax.dot_general(
            enc16, cbt16_ref[:, t * CB_TILE:(t + 1) * CB_TILE],
            (((1,), (0,)), ((), ())), preferred_element_type=_F32)
        d = enc_sq + cb_sq - 2.0 * sc
        m = jnp.min(d, axis=1, keepdims=True)
        col = jax.lax.broadcasted_iota(jnp.int32, (rows, CB_TILE), 1)
        a = jnp.min(jnp.where(d == m, col + t * CB_TILE, TOKEN_CLASS),
                    axis=1, keepdims=True)
        upd = m < best
        best = jnp.where(upd, m, best)
        bidx = jnp.where(upd, a, bidx)

    idx_ref[...] = bidx

    quant = jnp.zeros((rows, TOKEN_DIM), _F32)
    for t in range(n_tiles):
        col = jax.lax.broadcasted_iota(jnp.int32, (rows, CB_TILE), 1)
        oh = (bidx == col + t * CB_TILE).astype(_BF16)
        quant = quant + jax.lax.dot_general(
            oh, cb16_ref[t * CB_TILE:(t + 1) * CB_TILE, :],
            (((1,), (0,)), ((), ())), preferred_element_type=_F32)

    psum = jnp.sum((quant - enc) ** 2)

    @pl.when(step == 0)
    def _init():
        elat_ref[...] = jnp.zeros_like(elat_ref)

    elat_ref[0, 0, :] = elat_ref[0, 0, :] + psum

    q3 = quant.reshape(bblk, TOKEN_NUM, TOKEN_DIM)
    qt = jnp.swapaxes(q3, 1, 2).reshape(bblk * TOKEN_DIM, TOKEN_NUM)
    p = _mm(qt, w_dtok_ref[...]) + b_dtok_ref[...]
    p3 = p.reshape(bblk, TOKEN_DIM, NUM_JOINTS)
    p = jnp.swapaxes(p3, 1, 2).reshape(bblk * NUM_JOINTS, TOKEN_DIM)
    df = _mm(p, w_dstart_ref[...]) + b_dstart_ref[...]

    for i in range(N_MIX):
        df = _mixer(
            df, bblk, NUM_JOINTS,
            dec_ln1g_ref[i], dec_ln1b_ref[i],
            dec_tw1_ref[i], dec_tb1_ref[i], dec_tw2_ref[i], dec_tb2_ref[i],
            dec_ln2g_ref[i], dec_ln2b_ref[i],
            dec_cw1_ref[i], dec_cb1_ref[i], dec_cw2_ref[i], dec_cb2_ref[i])

    df = _ln(df, dec_lnfg_ref[...], dec_lnfb_ref[...])
    rec = _mm(df, w_rec_ref[...]) + b_rec_ref[...]
    rec_ref[...] = rec.reshape(bblk, NUM_JOINTS, INPUT_DIM)


def _row(x):
    return x.reshape(1, -1)


def _xla_ln(x, g, b, eps=1e-5):
    m = jnp.mean(x, axis=-1, keepdims=True)
    v = jnp.mean((x - m) ** 2, axis=-1, keepdims=True)
    return (x - m) / jnp.sqrt(v + eps) * g + b


def _xla_mlp(x, W1, b1, W2, b2):
    h = jax.nn.gelu(x @ W1 + b1, approximate=False)
    return h @ W2 + b2


def _xla_mixer(x, p, i):
    y = _xla_ln(x, p['ln1_g'][i], p['ln1_b'][i])
    y = jnp.swapaxes(y, 1, 2)
    y = _xla_mlp(y, p['tW1'][i], p['tb1'][i], p['tW2'][i], p['tb2'][i])
    y = jnp.swapaxes(y, 1, 2)
    z = _xla_ln(x + y, p['ln2_g'][i], p['ln2_b'][i])
    z = _xla_mlp(z, p['cW1'][i], p['cb1'][i], p['cW2'][i], p['cb2'][i])
    return x + y + z


def kernel(pose, params, codebook):
    bs = pose.shape[0]
    p = params
    bf = lambda x: x.astype(_BF16)

    f = pose @ p['W_start'] + p['b_start']
    for i in range(N_MIX):
        f = _xla_mixer(f, p['enc'], i)
    f = _xla_ln(f, p['enc_lnf_g'], p['enc_lnf_b'])
    f = jnp.swapaxes(f, 1, 2)
    f = f @ p['W_tok'] + p['b_tok']
    f = jnp.swapaxes(f, 1, 2)
    f = f @ p['W_feat'] + p['b_feat']
    enc = f.reshape(-1, TOKEN_DIM)
    n_rows = enc.shape[0]
    rblk = BBLK * TOKEN_NUM
    n_steps = n_rows // rblk

    dec = p['dec']
    consts = [
        jnp.sum(codebook ** 2, axis=1).reshape(1, TOKEN_CLASS),
        bf(codebook.T), bf(codebook),
        bf(p['W_dtok']), _row(p['b_dtok']),
        bf(p['W_dstart']), _row(p['b_dstart']),
        dec['ln1_g'], dec['ln1_b'], bf(dec['tW1']), dec['tb1'],
        bf(dec['tW2']), dec['tb2'], dec['ln2_g'], dec['ln2_b'],
        bf(dec['cW1']), dec['cb1'], bf(dec['cW2']), dec['cb2'],
        _row(p['dec_lnf_g']), _row(p['dec_lnf_b']),
        bf(p['W_rec']), _row(p['b_rec']),
    ]
    cs = lambda x: pl.BlockSpec(x.shape, lambda i, _n=x.ndim: (0,) * _n)

    rec, idx2d, elat = pl.pallas_call(
        _vqdec_kernel,
        grid=(n_steps,),
        in_specs=[pl.BlockSpec((rblk, TOKEN_DIM), lambda i: (i, 0))]
                 + [cs(x) for x in consts],
        out_specs=(
            pl.BlockSpec((BBLK, NUM_JOINTS, INPUT_DIM), lambda i: (i, 0, 0)),
            pl.BlockSpec((rblk, 1), lambda i: (i, 0)),
            pl.BlockSpec((1, 1, 128), lambda i: (0, 0, 0)),
        ),
        out_shape=(
            jax.ShapeDtypeStruct((bs, NUM_JOINTS, INPUT_DIM), _F32),
            jax.ShapeDtypeStruct((n_rows, 1), jnp.int32),
            jax.ShapeDtypeStruct((1, 1, 128), _F32),
        ),
    )(enc, *consts)

    idx = idx2d.reshape(n_rows)
    e_latent_loss = elat[0, 0, 0] / jnp.float32(n_rows * TOKEN_DIM)
    return rec, idx, e_latent_loss


# speedup vs baseline: 1.4308x; 1.0052x over previous
import jax
import jax.numpy as jnp
from jax.experimental import pallas as pl

NUM_JOINTS = 24
INPUT_DIM = 9
HID = 512
N_MIX = 4
TOKEN_NUM = 34
TOKEN_CLASS = 2048
TOKEN_DIM = 512
CB_TILE = 512

BBLK = 64

_BF16 = jnp.bfloat16
_F32 = jnp.float32


def _mm(a, b16):
    return jax.lax.dot_general(
        a.astype(_BF16) if a.dtype != _BF16 else a, b16,
        (((1,), (0,)), ((), ())), preferred_element_type=_F32)


def _ln(x, g, b, eps=1e-5):
    m = jnp.mean(x, axis=-1, keepdims=True)
    v = jnp.mean((x - m) ** 2, axis=-1, keepdims=True)
    return (x - m) / jnp.sqrt(v + eps) * g + b


def _gelu(x):
    return 0.5 * x * (1.0 + jax.lax.erf(x * jnp.float32(0.7071067811865476)))


def _token_mix(y2d, bblk, t_in, w1, b1, w2, b2):
    y3 = y2d.reshape(bblk, t_in, HID)
    yt = jnp.swapaxes(y3, 1, 2).reshape(bblk * HID, t_in)
    h = _gelu(_mm(yt, w1) + b1)
    z = _mm(h, w2) + b2
    t_out = z.shape[-1]
    z3 = z.reshape(bblk, HID, t_out)
    return jnp.swapaxes(z3, 1, 2).reshape(bblk * t_out, HID)


def _mixer(x2d, bblk, t, ln1_g, ln1_b, tw1, tb1, tw2, tb2,
           ln2_g, ln2_b, cw1, cb1, cw2, cb2):
    y = _ln(x2d, ln1_g, ln1_b)
    y = _token_mix(y, bblk, t, tw1, tb1, tw2, tb2)
    x1 = x2d + y
    z = _ln(x1, ln2_g, ln2_b)
    z = _mm(z, cw1) + cb1
    z = _gelu(z)
    z = _mm(z, cw2) + cb2
    return x1 + z


def _vqdec_kernel(
    enc_ref, cbsq_ref, cbt16_ref, cb16_ref,
    w_dtok_ref, b_dtok_ref, w_dstart_ref, b_dstart_ref,
    dec_ln1g_ref, dec_ln1b_ref, dec_tw1_ref, dec_tb1_ref, dec_tw2_ref,
    dec_tb2_ref, dec_ln2g_ref, dec_ln2b_ref, dec_cw1_ref, dec_cb1_ref,
    dec_cw2_ref, dec_cb2_ref,
    dec_lnfg_ref, dec_lnfb_ref,
    w_rec_ref, b_rec_ref,
    rec_ref, idx_ref, elat_ref,
):
    step = pl.program_id(0)
    rows = enc_ref.shape[0]
    bblk = rows // TOKEN_NUM

    enc = enc_ref[...]
    enc16 = enc.astype(_BF16)
    enc_sq = jnp.sum(enc * enc, axis=1, keepdims=True)
    best = jnp.full((rows, 1), jnp.inf, _F32)
    bidx = jnp.zeros((rows, 1), jnp.int32)
    n_tiles = TOKEN_CLASS // CB_TILE
    for t in range(n_tiles):
        cb_sq = cbsq_ref[:, t * CB_TILE:(t + 1) * CB_TILE]
        sc = jax.lax.dot_general(
            enc16, cbt16_ref[:, t * CB_TILE:(t + 1) * CB_TILE],
            (((1,), (0,)), ((), ())), preferred_element_type=_F32)
        d = enc_sq + cb_sq - 2.0 * sc
        m = jnp.min(d, axis=1, keepdims=True)
        col = jax.lax.broadcasted_iota(jnp.int32, (rows, CB_TILE), 1)
        a = jnp.min(jnp.where(d == m, col + t * CB_TILE, TOKEN_CLASS),
                    axis=1, keepdims=True)
        upd = m < best
        best = jnp.where(upd, m, best)
        bidx = jnp.where(upd, a, bidx)

    idx_ref[...] = bidx

    quant = jnp.zeros((rows, TOKEN_DIM), _F32)
    for t in range(n_tiles):
        col = jax.lax.broadcasted_iota(jnp.int32, (rows, CB_TILE), 1)
        oh = (bidx == col + t * CB_TILE).astype(_BF16)
        quant = quant + jax.lax.dot_general(
            oh, cb16_ref[t * CB_TILE:(t + 1) * CB_TILE, :],
            (((1,), (0,)), ((), ())), preferred_element_type=_F32)

    psum = jnp.sum((quant - enc) ** 2)

    @pl.when(step == 0)
    def _init():
        elat_ref[...] = jnp.zeros_like(elat_ref)

    elat_ref[0, 0, :] = elat_ref[0, 0, :] + psum

    q3 = quant.reshape(bblk, TOKEN_NUM, TOKEN_DIM)
    qt = jnp.swapaxes(q3, 1, 2).reshape(bblk * TOKEN_DIM, TOKEN_NUM)
    p = _mm(qt, w_dtok_ref[...]) + b_dtok_ref[...]
    p3 = p.reshape(bblk, TOKEN_DIM, NUM_JOINTS)
    p = jnp.swapaxes(p3, 1, 2).reshape(bblk * NUM_JOINTS, TOKEN_DIM)
    df = _mm(p, w_dstart_ref[...]) + b_dstart_ref[...]

    for i in range(N_MIX):
        df = _mixer(
            df, bblk, NUM_JOINTS,
            dec_ln1g_ref[i], dec_ln1b_ref[i],
            dec_tw1_ref[i], dec_tb1_ref[i], dec_tw2_ref[i], dec_tb2_ref[i],
            dec_ln2g_ref[i], dec_ln2b_ref[i],
            dec_cw1_ref[i], dec_cb1_ref[i], dec_cw2_ref[i], dec_cb2_ref[i])

    df = _ln(df, dec_lnfg_ref[...], dec_lnfb_ref[...])
    rec = _mm(df, w_rec_ref[...]) + b_rec_ref[...]
    rec_ref[...] = rec.reshape(bblk, NUM_JOINTS, INPUT_DIM)


def _row(x):
    return x.reshape(1, -1)


def _xla_ln(x, g, b, eps=1e-5):
    m = jnp.mean(x, axis=-1, keepdims=True)
    v = jnp.mean((x - m) ** 2, axis=-1, keepdims=True)
    return (x - m) / jnp.sqrt(v + eps) * g + b


def _xla_mlp(x, W1, b1, W2, b2):
    h = jax.nn.gelu(x @ W1 + b1, approximate=False)
    return h @ W2 + b2


def _xla_mixer(x, p, i):
    y = _xla_ln(x, p['ln1_g'][i], p['ln1_b'][i])
    y = jnp.swapaxes(y, 1, 2)
    y = _xla_mlp(y, p['tW1'][i], p['tb1'][i], p['tW2'][i], p['tb2'][i])
    y = jnp.swapaxes(y, 1, 2)
    z = _xla_ln(x + y, p['ln2_g'][i], p['ln2_b'][i])
    z = _xla_mlp(z, p['cW1'][i], p['cb1'][i], p['cW2'][i], p['cb2'][i])
    return x + y + z


def kernel(pose, params, codebook):
    bs = pose.shape[0]
    p = params
    bf = lambda x: x.astype(_BF16)

    f = pose @ p['W_start'] + p['b_start']
    for i in range(N_MIX):
        f = _xla_mixer(f, p['enc'], i)
    f = _xla_ln(f, p['enc_lnf_g'], p['enc_lnf_b'])
    f = jnp.swapaxes(f, 1, 2)
    f = f @ p['W_tok'] + p['b_tok']
    f = jnp.swapaxes(f, 1, 2)
    f = f @ p['W_feat'] + p['b_feat']
    enc = f.reshape(-1, TOKEN_DIM)
    n_rows = enc.shape[0]
    rblk = BBLK * TOKEN_NUM
    n_steps = n_rows // rblk

    dec = p['dec']
    consts = [
        jnp.sum(codebook ** 2, axis=1).reshape(1, TOKEN_CLASS),
        bf(codebook.T), bf(codebook),
        bf(p['W_dtok']), _row(p['b_dtok']),
        bf(p['W_dstart']), _row(p['b_dstart']),
        dec['ln1_g'], dec['ln1_b'], bf(dec['tW1']), dec['tb1'],
        bf(dec['tW2']), dec['tb2'], dec['ln2_g'], dec['ln2_b'],
        bf(dec['cW1']), dec['cb1'], bf(dec['cW2']), dec['cb2'],
        _row(p['dec_lnf_g']), _row(p['dec_lnf_b']),
        bf(p['W_rec']), _row(p['b_rec']),
    ]
    cs = lambda x: pl.BlockSpec(x.shape, lambda i, _n=x.ndim: (0,) * _n)

    rec, idx2d, elat = pl.pallas_call(
        _vqdec_kernel,
        grid=(n_steps,),
        in_specs=[pl.BlockSpec((rblk, TOKEN_DIM), lambda i: (i, 0))]
                 + [cs(x) for x in consts],
        out_specs=(
            pl.BlockSpec((BBLK, NUM_JOINTS, INPUT_DIM), lambda i: (i, 0, 0)),
            pl.BlockSpec((rblk, 1), lambda i: (i, 0)),
            pl.BlockSpec((1, 1, 128), lambda i: (0, 0, 0)),
        ),
        out_shape=(
            jax.ShapeDtypeStruct((bs, NUM_JOINTS, INPUT_DIM), _F32),
            jax.ShapeDtypeStruct((n_rows, 1), jnp.int32),
            jax.ShapeDtypeStruct((1, 1, 128), _F32),
        ),
    )(enc, *consts)

    idx = idx2d.reshape(n_rows)
    e_latent_loss = elat[0, 0, 0] / jnp.float32(n_rows * TOKEN_DIM)
    return rec, idx, e_latent_loss
